# initial kernel scaffold (unmeasured)
import jax
import jax.numpy as jnp
from jax import lax
from jax.experimental import pallas as pl
from jax.experimental.pallas import tpu as pltpu


def kernel(
    x,
):
    def body(*refs):
        pass

    out_shape = jax.ShapeDtypeStruct(..., jnp.float32)
    return pl.pallas_call(body, out_shape=out_shape)(...)


# baseline (device time: 21129 ns/iter reference)
import jax
import jax.numpy as jnp
from jax import lax
from jax.experimental import pallas as pl
from jax.experimental.pallas import tpu as pltpu

N_DEV = 4


def kernel(x):
    m, n = x.shape
    blk = m // N_DEV

    def body(x_ref, out_ref, xb_ref, rs_ref, rs_send, rs_recv, ag_send, ag_recv):
        me = lax.axis_index("i")

        barrier_sem = pltpu.get_barrier_semaphore()
        for j in range(1, N_DEV):
            pl.semaphore_signal(
                barrier_sem, inc=1,
                device_id=((me + j) % N_DEV,),
                device_id_type=pl.DeviceIdType.MESH,
            )
        pl.semaphore_wait(barrier_sem, N_DEV - 1)

        xb_ref[...] = x_ref[...].astype(jnp.bfloat16)

        rs_rdmas = []
        for j in range(1, N_DEV):
            tgt = (me + j) % N_DEV
            rdma = pltpu.make_async_remote_copy(
                src_ref=xb_ref.at[pl.ds(tgt * blk, blk), :],
                dst_ref=rs_ref.at[j],
                send_sem=rs_send.at[j],
                recv_sem=rs_recv.at[j],
                device_id=(tgt,),
                device_id_type=pl.DeviceIdType.MESH,
            )
            rdma.start()
            rs_rdmas.append(rdma)
        for rdma in rs_rdmas:
            rdma.wait()

        acc = x_ref[pl.ds(me * blk, blk), :]
        for j in range(1, N_DEV):
            acc = acc + rs_ref[j].astype(jnp.float32)
        out_ref[pl.ds(me * blk, blk), :] = acc.astype(jnp.bfloat16)

        ag_rdmas = []
        for j in range(1, N_DEV):
            tgt = (me + j) % N_DEV
            rdma = pltpu.make_async_remote_copy(
                src_ref=out_ref.at[pl.ds(me * blk, blk), :],
                dst_ref=out_ref.at[pl.ds(me * blk, blk), :],
                send_sem=ag_send.at[j],
                recv_sem=ag_recv.at[j],
                device_id=(tgt,),
                device_id_type=pl.DeviceIdType.MESH,
            )
            rdma.start()
            ag_rdmas.append(rdma)
        for rdma in ag_rdmas:
            rdma.wait()

    return pl.pallas_call(
        body,
        out_shape=jax.ShapeDtypeStruct((m, n), jnp.bfloat16),
        in_specs=[pl.BlockSpec(memory_space=pltpu.VMEM)],
        out_specs=pl.BlockSpec(memory_space=pltpu.VMEM),
        scratch_shapes=[
            pltpu.VMEM((m, n), jnp.bfloat16),
            pltpu.VMEM((N_DEV, blk, n), jnp.bfloat16),
            pltpu.SemaphoreType.DMA((N_DEV,)),
            pltpu.SemaphoreType.DMA((N_DEV,)),
            pltpu.SemaphoreType.DMA((N_DEV,)),
            pltpu.SemaphoreType.DMA((N_DEV,)),
        ],
        compiler_params=pltpu.CompilerParams(collective_id=0),
    )(x)


# device time: 21082 ns/iter; 1.0022x vs baseline; 1.0022x over previous
import jax
import jax.numpy as jnp
from jax import lax
from jax.experimental import pallas as pl
from jax.experimental.pallas import tpu as pltpu

N_DEV = 4


def kernel(x):
    m, n = x.shape
    blk = m // N_DEV

    def body(x_ref, out_ref, xb_ref, rs_ref, rs_send, rs_recv, ag_send, ag_recv):
        me = lax.axis_index("i")

        xb_ref[...] = x_ref[...].astype(jnp.bfloat16)

        barrier_sem = pltpu.get_barrier_semaphore()
        for j in range(1, N_DEV):
            pl.semaphore_signal(
                barrier_sem, inc=1,
                device_id=((me + j) % N_DEV,),
                device_id_type=pl.DeviceIdType.MESH,
            )
        pl.semaphore_wait(barrier_sem, N_DEV - 1)

        rs_rdmas = []
        for j in (2, 1, 3):
            tgt = (me + j) % N_DEV
            rdma = pltpu.make_async_remote_copy(
                src_ref=xb_ref.at[pl.ds(tgt * blk, blk), :],
                dst_ref=rs_ref.at[j],
                send_sem=rs_send.at[j],
                recv_sem=rs_recv.at[j],
                device_id=(tgt,),
                device_id_type=pl.DeviceIdType.MESH,
            )
            rdma.start()
            rs_rdmas.append(rdma)
        for rdma in rs_rdmas:
            rdma.wait()

        acc = x_ref[pl.ds(me * blk, blk), :]
        for j in range(1, N_DEV):
            acc = acc + rs_ref[j].astype(jnp.float32)
        out_ref[pl.ds(me * blk, blk), :] = acc.astype(jnp.bfloat16)

        ag_rdmas = []
        for j in (2, 1, 3):
            tgt = (me + j) % N_DEV
            rdma = pltpu.make_async_remote_copy(
                src_ref=out_ref.at[pl.ds(me * blk, blk), :],
                dst_ref=out_ref.at[pl.ds(me * blk, blk), :],
                send_sem=ag_send.at[j],
                recv_sem=ag_recv.at[j],
                device_id=(tgt,),
                device_id_type=pl.DeviceIdType.MESH,
            )
            rdma.start()
            ag_rdmas.append(rdma)
        for rdma in ag_rdmas:
            rdma.wait()

    return pl.pallas_call(
        body,
        out_shape=jax.ShapeDtypeStruct((m, n), jnp.bfloat16),
        in_specs=[pl.BlockSpec(memory_space=pltpu.VMEM)],
        out_specs=pl.BlockSpec(memory_space=pltpu.VMEM),
        scratch_shapes=[
            pltpu.VMEM((m, n), jnp.bfloat16),
            pltpu.VMEM((N_DEV, blk, n), jnp.bfloat16),
            pltpu.SemaphoreType.DMA((N_DEV,)),
            pltpu.SemaphoreType.DMA((N_DEV,)),
            pltpu.SemaphoreType.DMA((N_DEV,)),
            pltpu.SemaphoreType.DMA((N_DEV,)),
        ],
        compiler_params=pltpu.CompilerParams(collective_id=0),
    )(x)


# device time: 19075 ns/iter; 1.1077x vs baseline; 1.1052x over previous
import jax
import jax.numpy as jnp
from jax import lax
from jax.experimental import pallas as pl
from jax.experimental.pallas import tpu as pltpu

N_DEV = 4
SEG = 2


def kernel(x):
    m, n = x.shape
    blk = m // N_DEV
    sub = blk // SEG

    def body(x_ref, out_ref, xb_ref, rs_ref, rs_send, rs_recv, ag_send, ag_recv):
        me = lax.axis_index("i")

        xb_ref[...] = x_ref[...].astype(jnp.bfloat16)

        barrier_sem = pltpu.get_barrier_semaphore()
        for j in range(1, N_DEV):
            pl.semaphore_signal(
                barrier_sem, inc=1,
                device_id=((me + j) % N_DEV,),
                device_id_type=pl.DeviceIdType.MESH,
            )
        pl.semaphore_wait(barrier_sem, N_DEV - 1)

        rs_rdmas = {}
        for r in range(SEG):
            for j in (2, 1, 3):
                tgt = (me + j) % N_DEV
                rdma = pltpu.make_async_remote_copy(
                    src_ref=xb_ref.at[pl.ds(tgt * blk + r * sub, sub), :],
                    dst_ref=rs_ref.at[j, pl.ds(r * sub, sub), :],
                    send_sem=rs_send.at[r, j],
                    recv_sem=rs_recv.at[r, j],
                    device_id=(tgt,),
                    device_id_type=pl.DeviceIdType.MESH,
                )
                rdma.start()
                rs_rdmas[(r, j)] = rdma

        ag_rdmas = []
        for r in range(SEG):
            for j in (2, 1, 3):
                rs_rdmas[(r, j)].wait()
            rows = pl.ds(me * blk + r * sub, sub)
            acc = x_ref[rows, :]
            for j in range(1, N_DEV):
                acc = acc + rs_ref[j, pl.ds(r * sub, sub), :].astype(jnp.float32)
            out_ref[rows, :] = acc.astype(jnp.bfloat16)
            for j in (2, 1, 3):
                tgt = (me + j) % N_DEV
                rdma = pltpu.make_async_remote_copy(
                    src_ref=out_ref.at[rows, :],
                    dst_ref=out_ref.at[rows, :],
                    send_sem=ag_send.at[r, j],
                    recv_sem=ag_recv.at[r, j],
                    device_id=(tgt,),
                    device_id_type=pl.DeviceIdType.MESH,
                )
                rdma.start()
                ag_rdmas.append(rdma)

        for rdma in ag_rdmas:
            rdma.wait()

    return pl.pallas_call(
        body,
        out_shape=jax.ShapeDtypeStruct((m, n), jnp.bfloat16),
        in_specs=[pl.BlockSpec(memory_space=pltpu.VMEM)],
        out_specs=pl.BlockSpec(memory_space=pltpu.VMEM),
        scratch_shapes=[
            pltpu.VMEM((m, n), jnp.bfloat16),
            pltpu.VMEM((N_DEV, blk, n), jnp.bfloat16),
            pltpu.SemaphoreType.DMA((SEG, N_DEV)),
            pltpu.SemaphoreType.DMA((SEG, N_DEV)),
            pltpu.SemaphoreType.DMA((SEG, N_DEV)),
            pltpu.SemaphoreType.DMA((SEG, N_DEV)),
        ],
        compiler_params=pltpu.CompilerParams(collective_id=0),
    )(x)


# device time: 17849 ns/iter; 1.1838x vs baseline; 1.0687x over previous
import jax
import jax.numpy as jnp
from jax import lax
from jax.experimental import pallas as pl
from jax.experimental.pallas import tpu as pltpu

N_DEV = 4
SEG = 2


def kernel(x):
    m, n = x.shape
    blk = m // N_DEV
    sub = blk // SEG

    def body(x_ref, out_ref, xb_ref, rs_ref, rs_send, rs_recv, ag_send, ag_recv):
        me = lax.axis_index("i")

        barrier_sem = pltpu.get_barrier_semaphore()
        for j in range(1, N_DEV):
            pl.semaphore_signal(
                barrier_sem, inc=1,
                device_id=((me + j) % N_DEV,),
                device_id_type=pl.DeviceIdType.MESH,
            )
        xb_ref[...] = x_ref[...].astype(jnp.bfloat16)
        pl.semaphore_wait(barrier_sem, N_DEV - 1)

        rs_rdmas = {}
        for r in range(SEG):
            for j in (2, 1, 3):
                tgt = (me + j) % N_DEV
                rdma = pltpu.make_async_remote_copy(
                    src_ref=xb_ref.at[pl.ds(tgt * blk + r * sub, sub), :],
                    dst_ref=rs_ref.at[j, pl.ds(r * sub, sub), :],
                    send_sem=rs_send.at[r, j],
                    recv_sem=rs_recv.at[r, j],
                    device_id=(tgt,),
                    device_id_type=pl.DeviceIdType.MESH,
                )
                rdma.start()
                rs_rdmas[(r, j)] = rdma

        ag_rdmas = []
        for r in range(SEG):
            for j in (2, 1, 3):
                rs_rdmas[(r, j)].wait()
            rows = pl.ds(me * blk + r * sub, sub)
            acc = x_ref[rows, :]
            for j in range(1, N_DEV):
                acc = acc + rs_ref[j, pl.ds(r * sub, sub), :].astype(jnp.float32)
            out_ref[rows, :] = acc.astype(jnp.bfloat16)
            for j in (2, 1, 3):
                tgt = (me + j) % N_DEV
                rdma = pltpu.make_async_remote_copy(
                    src_ref=out_ref.at[rows, :],
                    dst_ref=out_ref.at[rows, :],
                    send_sem=ag_send.at[r, j],
                    recv_sem=ag_recv.at[r, j],
                    device_id=(tgt,),
                    device_id_type=pl.DeviceIdType.MESH,
                )
                rdma.start()
                ag_rdmas.append(rdma)

        for rdma in ag_rdmas:
            rdma.wait()

    return pl.pallas_call(
        body,
        out_shape=jax.ShapeDtypeStruct((m, n), jnp.bfloat16),
        in_specs=[pl.BlockSpec(memory_space=pltpu.VMEM)],
        out_specs=pl.BlockSpec(memory_space=pltpu.VMEM),
        scratch_shapes=[
            pltpu.VMEM((m, n), jnp.bfloat16),
            pltpu.VMEM((N_DEV, blk, n), jnp.bfloat16),
            pltpu.SemaphoreType.DMA((SEG, N_DEV)),
            pltpu.SemaphoreType.DMA((SEG, N_DEV)),
            pltpu.SemaphoreType.DMA((SEG, N_DEV)),
            pltpu.SemaphoreType.DMA((SEG, N_DEV)),
        ],
        compiler_params=pltpu.CompilerParams(collective_id=0),
    )(x)
